# baseline (device time: 14414 ns/iter reference)
import jax
import jax.numpy as jnp
from jax import lax
from jax.experimental import pallas as pl
from jax.experimental.pallas import tpu as pltpu

N_DEV = 4


def kernel(x, Wq, K_ext, V_ext, Wo):
    B, Sq, D = x.shape
    _, Skv, Hl, Dh = K_ext.shape
    Dout = Wo.shape[1]
    Hd = Hl * Dh
    Dhalf = Dout // 2
    M = B * Sq
    scale = 1.0 / (Dh ** 0.5)
    bf16 = jnp.bfloat16
    f32 = jnp.float32

    my_i_outer = lax.axis_index("i")

    def body(q_ref, k_ref, v_ref, wo_ref, out_ref,
             send_ref, acc_ref, recv_ref, send_sems, recv_sems):
        my_i = lax.axis_index("i")
        part_a = my_i ^ 1
        part_b = my_i ^ 3

        barrier_sem = pltpu.get_barrier_semaphore()
        for nbr in (part_a, part_b):
            pl.semaphore_signal(
                barrier_sem, inc=1,
                device_id=(nbr,), device_id_type=pl.DeviceIdType.MESH,
            )

        def exchange(slot, src, tgt):
            rdma = pltpu.make_async_remote_copy(
                src_ref=src,
                dst_ref=recv_ref.at[slot],
                send_sem=send_sems.at[slot],
                recv_sem=recv_sems.at[slot],
                device_id=(tgt,),
                device_id_type=pl.DeviceIdType.MESH,
            )
            rdma.start()
            return rdma

        halves = [[None, None] for _ in range(B)]
        s1 = [[None, None] for _ in range(B)]
        for b in range(B):
            ctx_cols = []
            for h in range(Hl):
                qh = q_ref[b * Sq:(b + 1) * Sq, h * Dh:(h + 1) * Dh]
                kh = k_ref[b, :, h * Dh:(h + 1) * Dh]
                vh = v_ref[b, :, h * Dh:(h + 1) * Dh]
                s = lax.dot_general(
                    qh, kh, (((1,), (1,)), ((), ())),
                    preferred_element_type=f32,
                ) * scale
                e = jnp.exp(s)
                r = 1.0 / jnp.sum(e, axis=1, keepdims=True)
                pv = jnp.dot(e.astype(bf16), vh, preferred_element_type=f32)
                ctx_cols.append(pv * r)
            ctx_b = jnp.concatenate(ctx_cols, axis=1).astype(bf16)

            ha = jnp.dot(ctx_b, wo_ref[:, :Dhalf], preferred_element_type=f32)
            send_ref[b * 2 + 0] = ha.astype(bf16)
            if b == 0:
                pl.semaphore_wait(barrier_sem, 2)
            s1[b][0] = exchange(b * 2 + 0, send_ref.at[b * 2 + 0], part_a)
            hb = jnp.dot(ctx_b, wo_ref[:, Dhalf:], preferred_element_type=f32)
            send_ref[b * 2 + 1] = hb.astype(bf16)
            s1[b][1] = exchange(b * 2 + 1, send_ref.at[b * 2 + 1], part_b)
            halves[b] = [ha, hb]

        accs = [[None, None] for _ in range(B)]
        s2 = [[None, None] for _ in range(B)]
        for b in range(B):
            s1[b][0].wait()
            acc_a = halves[b][0] + recv_ref[b * 2 + 0].astype(f32)
            accs[b][0] = acc_a
            acc_ref[b * 2 + 0] = acc_a.astype(bf16)
            s2[b][0] = exchange(4 + b * 2 + 0, acc_ref.at[b * 2 + 0], part_b)
            s1[b][1].wait()
            acc_b = halves[b][1] + recv_ref[b * 2 + 1].astype(f32)
            accs[b][1] = acc_b
            acc_ref[b * 2 + 1] = acc_b.astype(bf16)
            s2[b][1] = exchange(4 + b * 2 + 1, acc_ref.at[b * 2 + 1], part_a)

        for b in range(B):
            s2[b][0].wait()
            out_ref[b, :, :Dhalf] = (
                accs[b][0] + recv_ref[4 + b * 2 + 0].astype(f32)
            ).astype(bf16)
            s2[b][1].wait()
            out_ref[b, :, Dhalf:] = (
                accs[b][1] + recv_ref[4 + b * 2 + 1].astype(f32)
            ).astype(bf16)

    q16 = jnp.dot(
        x.reshape(M, D),
        lax.dynamic_slice_in_dim(Wq, my_i_outer * Hd, Hd, 1),
        preferred_element_type=jnp.float32,
    ).astype(bf16)

    return pl.pallas_call(
        body,
        out_shape=jax.ShapeDtypeStruct((B, Sq, Dout), bf16),
        in_specs=[pl.BlockSpec(memory_space=pltpu.VMEM)] * 4,
        out_specs=pl.BlockSpec(memory_space=pltpu.VMEM),
        scratch_shapes=[
            pltpu.VMEM((4, Sq, Dhalf), bf16),
            pltpu.VMEM((4, Sq, Dhalf), bf16),
            pltpu.VMEM((8, Sq, Dhalf), bf16),
            pltpu.SemaphoreType.DMA((8,)),
            pltpu.SemaphoreType.DMA((8,)),
        ],
        compiler_params=pltpu.CompilerParams(collective_id=0),
    )(
        q16,
        K_ext.reshape(B, Skv, Hd).astype(bf16),
        V_ext.reshape(B, Skv, Hd).astype(bf16),
        lax.dynamic_slice_in_dim(Wo, my_i_outer * Hd, Hd, 0).astype(bf16),
    )


# device time: 14011 ns/iter; 1.0288x vs baseline; 1.0288x over previous
import jax
import jax.numpy as jnp
from jax import lax
from jax.experimental import pallas as pl
from jax.experimental.pallas import tpu as pltpu

N_DEV = 4


def kernel(x, Wq, K_ext, V_ext, Wo):
    B, Sq, D = x.shape
    _, Skv, Hl, Dh = K_ext.shape
    Dout = Wo.shape[1]
    Hd = Hl * Dh
    Dhalf = Dout // 2
    M = B * Sq
    scale = 1.0 / (Dh ** 0.5)
    bf16 = jnp.bfloat16
    f32 = jnp.float32

    my_i_outer = lax.axis_index("i")

    def body(q_ref, k_ref, v_ref, wo_ref, out_ref,
             send_ref, acc_ref, recv_ref, send_sems, recv_sems):
        my_i = lax.axis_index("i")
        part_a = my_i ^ 1
        part_b = my_i ^ 3

        barrier_sem = pltpu.get_barrier_semaphore()
        for nbr in (part_a, part_b):
            pl.semaphore_signal(
                barrier_sem, inc=1,
                device_id=(nbr,), device_id_type=pl.DeviceIdType.MESH,
            )

        def exchange(slot, src, tgt):
            rdma = pltpu.make_async_remote_copy(
                src_ref=src,
                dst_ref=recv_ref.at[slot],
                send_sem=send_sems.at[slot],
                recv_sem=recv_sems.at[slot],
                device_id=(tgt,),
                device_id_type=pl.DeviceIdType.MESH,
            )
            rdma.start()
            return rdma

        halves = [[None, None] for _ in range(B)]
        s1 = [[None, None] for _ in range(B)]
        for b in range(B):
            ctx_cols = []
            for h in range(Hl):
                qh = q_ref[b * Sq:(b + 1) * Sq, h * Dh:(h + 1) * Dh]
                kh = k_ref[b, :, h * Dh:(h + 1) * Dh]
                vh = v_ref[b, :, h * Dh:(h + 1) * Dh]
                s = lax.dot_general(
                    qh, kh, (((1,), (1,)), ((), ())),
                    preferred_element_type=f32,
                ) * scale
                e = jnp.exp(s)
                r = 1.0 / jnp.sum(e, axis=1, keepdims=True)
                pv = jnp.dot(e.astype(bf16), vh, preferred_element_type=f32)
                ctx_cols.append(pv * r)
            ctx_b = jnp.concatenate(ctx_cols, axis=1).astype(bf16)

            ha = jnp.dot(ctx_b, wo_ref[:, :Dhalf], preferred_element_type=f32)
            hb = jnp.dot(ctx_b, wo_ref[:, Dhalf:], preferred_element_type=f32)
            halves[b] = [ha, hb]
            send_ref[b * 2 + 0] = ha.astype(bf16)
            send_ref[b * 2 + 1] = hb.astype(bf16)
            if b == 0:
                pl.semaphore_wait(barrier_sem, 2)
            s1[b][0] = exchange(b * 2 + 0, send_ref.at[b * 2 + 0], part_a)
            s1[b][1] = exchange(b * 2 + 1, send_ref.at[b * 2 + 1], part_b)

        accs = [[None, None] for _ in range(B)]
        s2 = [[None, None] for _ in range(B)]
        for b in range(B):
            s1[b][0].wait()
            acc_a = halves[b][0] + recv_ref[b * 2 + 0].astype(f32)
            accs[b][0] = acc_a
            acc_ref[b * 2 + 0] = acc_a.astype(bf16)
            s2[b][0] = exchange(4 + b * 2 + 0, acc_ref.at[b * 2 + 0], part_b)
            s1[b][1].wait()
            acc_b = halves[b][1] + recv_ref[b * 2 + 1].astype(f32)
            accs[b][1] = acc_b
            acc_ref[b * 2 + 1] = acc_b.astype(bf16)
            s2[b][1] = exchange(4 + b * 2 + 1, acc_ref.at[b * 2 + 1], part_a)

        for b in range(B):
            s2[b][0].wait()
            out_ref[b, :, :Dhalf] = (
                accs[b][0] + recv_ref[4 + b * 2 + 0].astype(f32)
            ).astype(bf16)
            s2[b][1].wait()
            out_ref[b, :, Dhalf:] = (
                accs[b][1] + recv_ref[4 + b * 2 + 1].astype(f32)
            ).astype(bf16)

    q16 = jnp.dot(
        x.reshape(M, D),
        lax.dynamic_slice_in_dim(Wq, my_i_outer * Hd, Hd, 1),
        preferred_element_type=jnp.float32,
    ).astype(bf16)

    return pl.pallas_call(
        body,
        out_shape=jax.ShapeDtypeStruct((B, Sq, Dout), bf16),
        in_specs=[pl.BlockSpec(memory_space=pltpu.VMEM)] * 4,
        out_specs=pl.BlockSpec(memory_space=pltpu.VMEM),
        scratch_shapes=[
            pltpu.VMEM((4, Sq, Dhalf), bf16),
            pltpu.VMEM((4, Sq, Dhalf), bf16),
            pltpu.VMEM((8, Sq, Dhalf), bf16),
            pltpu.SemaphoreType.DMA((8,)),
            pltpu.SemaphoreType.DMA((8,)),
        ],
        compiler_params=pltpu.CompilerParams(collective_id=0),
    )(
        q16,
        K_ext.reshape(B, Skv, Hd).astype(bf16),
        V_ext.reshape(B, Skv, Hd).astype(bf16),
        lax.dynamic_slice_in_dim(Wo, my_i_outer * Hd, Hd, 0).astype(bf16),
    )
